# baseline (speedup 1.0000x reference)
import jax
import jax.numpy as jnp
from jax.experimental import pallas as pl
from jax.experimental.pallas import tpu as pltpu

_EPS = 1e-5


def _ln(x, c_real, masked_out=True):
    cpad = x.shape[-1]
    if c_real == cpad:
        mean = jnp.mean(x, axis=-1, keepdims=True)
        xc = x - mean
        var = jnp.mean(xc * xc, axis=-1, keepdims=True)
        return xc * jax.lax.rsqrt(var + _EPS)
    inv = 1.0 / float(c_real)
    mean = jnp.sum(x, axis=-1, keepdims=True) * inv
    var = jnp.sum(x * x, axis=-1, keepdims=True) * inv - mean * mean
    r = jax.lax.rsqrt(var + _EPS)
    if masked_out:
        lane = jax.lax.broadcasted_iota(jnp.int32, x.shape, x.ndim - 1)
        mask = (lane < c_real).astype(jnp.float32)
        return (x - mean) * (r * mask)
    return (x - mean) * r


def _gelu2(x):
    return x + x * jax.lax.erf(x * 0.7071067811865476)


def _mlp_residual(t, w1, b1, w2h, b2, c):
    h = _ln(t, c, masked_out=False)
    h = jnp.dot(h, w1, preferred_element_type=jnp.float32) + b1
    h = _gelu2(h)
    h = jnp.dot(h, w2h, preferred_element_type=jnp.float32) + b2
    return t + h


def _merge(t, g, grid_hw, mw, mb, c_out):
    cp = t.shape[-1]
    z = t.reshape(g * grid_hw * grid_hw // 2, 2 * cp)
    z = z.reshape(g, grid_hw // 2, 2, grid_hw // 2, 2 * cp)
    e = z[:, :, 0].reshape(g * (grid_hw // 2) ** 2, 2 * cp)
    o = z[:, :, 1].reshape(g * (grid_hw // 2) ** 2, 2 * cp)
    y = (jnp.dot(e, mw[: 2 * cp], preferred_element_type=jnp.float32)
         + jnp.dot(o, mw[2 * cp:], preferred_element_type=jnp.float32) + mb)
    return _ln(y, c_out)


def _backbone_kernel(g, x_ref, ew_ref, eb_ref,
                     w10_ref, b10_ref, w20_ref, b20_ref, mw0_ref, mb0_ref,
                     w11_ref, b11_ref, w21_ref, b21_ref, mw1_ref, mb1_ref,
                     w12_ref, b12_ref, w22_ref, b22_ref, mw2_ref, mb2_ref,
                     w13_ref, b13_ref, w23_ref, b23_ref,
                     o0_ref, o1_ref, o2_ref, o3_ref):
    xp = x_ref[...].reshape(g * 3136, 48).astype(jnp.float32)
    y = jnp.dot(xp, ew_ref[...], preferred_element_type=jnp.float32) + eb_ref[...]
    t = _ln(y, 96)
    t = _mlp_residual(t, w10_ref[...], b10_ref[...], w20_ref[...], b20_ref[...], 96)

    t = _merge(t, g, 56, mw0_ref[...], mb0_ref[...], 192)
    t = _mlp_residual(t, w11_ref[...], b11_ref[...], w21_ref[...], b21_ref[...], 192)

    t = _merge(t, g, 28, mw1_ref[...], mb1_ref[...], 384)
    t = _mlp_residual(t, w12_ref[...], b12_ref[...], w22_ref[...], b22_ref[...], 384)

    n = _ln(t, 384)
    o0_ref[...] = n.reshape(g, 196, 384).astype(o0_ref.dtype)
    a = n.reshape(g, 7, 2, 14, 384)
    r = jnp.maximum(a[:, :, 0], a[:, :, 1])
    b4 = r.reshape(g, 7, 7, 2, 384)
    p = jnp.maximum(b4[:, :, :, 0], b4[:, :, :, 1])
    o1_ref[...] = p.reshape(g, 49, 384).astype(o1_ref.dtype)

    t = _merge(t, g, 14, mw2_ref[...], mb2_ref[...], 768)
    t = _mlp_residual(t, w13_ref[...], b13_ref[...], w23_ref[...], b23_ref[...], 768)

    n = _ln(t, 768).reshape(g, 49, 768)
    o2_ref[...] = n.astype(o2_ref.dtype)
    o3_ref[...] = jnp.max(n, axis=1, keepdims=True).astype(o3_ref.dtype)


def _const_spec(shape):
    nd = len(shape)
    return pl.BlockSpec(shape, lambda i: (0,) * nd)


def kernel(x, embed_w, embed_b,
           s0_fc1_w, s0_fc1_b, s0_fc2_w, s0_fc2_b, s0_merge_w, s0_merge_b,
           s1_fc1_w, s1_fc1_b, s1_fc2_w, s1_fc2_b, s1_merge_w, s1_merge_b,
           s2_fc1_w, s2_fc1_b, s2_fc2_w, s2_fc2_b, s2_merge_w, s2_merge_b,
           s3_fc1_w, s3_fc1_b, s3_fc2_w, s3_fc2_b):
    B = x.shape[0]
    f32 = jnp.float32
    g = 4

    xp = x.reshape(B, 3, 56, 4, 56, 4)
    xp = jnp.transpose(xp, (0, 2, 4, 3, 5, 1)).reshape(B, 3136, 48)
    xp = xp.astype(jnp.bfloat16)

    wz = lambda w: w
    bz = lambda b: b.reshape(1, -1).astype(f32)

    def bzp(b, c):
        b = b.reshape(1, -1).astype(f32)
        lane = jax.lax.broadcasted_iota(jnp.int32, b.shape, 1)
        return jnp.where(lane < c, b, 0.0)

    half = lambda w: (0.5 * w.astype(f32))
    img = lambda n, c: pl.BlockSpec((g, n, c), lambda i: (i, 0, 0))

    weights = [
        (wz(embed_w), (48, 128)), (bzp(embed_b, 96), (1, 128)),
        (wz(s0_fc1_w), (128, 256)), (bz(s0_fc1_b), (1, 256)),
        (half(s0_fc2_w), (256, 128)), (bzp(s0_fc2_b, 96), (1, 128)),
        (wz(s0_merge_w), (512, 256)), (bzp(s0_merge_b, 192), (1, 256)),
        (wz(s1_fc1_w), (256, 384)), (bz(s1_fc1_b), (1, 384)),
        (half(s1_fc2_w), (384, 256)), (bzp(s1_fc2_b, 192), (1, 256)),
        (wz(s1_merge_w), (1024, 384)), (bz(s1_merge_b), (1, 384)),
        (wz(s2_fc1_w), (384, 768)), (bz(s2_fc1_b), (1, 768)),
        (half(s2_fc2_w), (768, 384)), (bz(s2_fc2_b), (1, 384)),
        (wz(s2_merge_w), (1536, 768)), (bz(s2_merge_b), (1, 768)),
        (wz(s3_fc1_w), (768, 1536)), (bz(s3_fc1_b), (1, 1536)),
        (half(s3_fc2_w), (1536, 768)), (bz(s3_fc2_b), (1, 768)),
    ]

    o0, o1, o2, o3 = pl.pallas_call(
        lambda *a: _backbone_kernel(g, *a),
        out_shape=(jax.ShapeDtypeStruct((B, 196, 384), f32),
                   jax.ShapeDtypeStruct((B, 49, 384), f32),
                   jax.ShapeDtypeStruct((B, 49, 768), f32),
                   jax.ShapeDtypeStruct((B, 1, 768), f32)),
        grid=(B // g,),
        in_specs=[img(3136, 48)] + [_const_spec(s) for _, s in weights],
        out_specs=(img(196, 384), img(49, 384), img(49, 768), img(1, 768)),
        compiler_params=pltpu.CompilerParams(
            dimension_semantics=("parallel",)),
    )(xp, *[w for w, _ in weights])

    return [[o0, o1], [o2, o3]]


# speedup vs baseline: 1.1044x; 1.1044x over previous
import jax
import jax.numpy as jnp
from jax.experimental import pallas as pl
from jax.experimental.pallas import tpu as pltpu

_EPS = 1e-5


def _ln(x, c_real, masked_out=True):
    cpad = x.shape[-1]
    if c_real == cpad:
        mean = jnp.mean(x, axis=-1, keepdims=True)
        xc = x - mean
        var = jnp.mean(xc * xc, axis=-1, keepdims=True)
        return xc * jax.lax.rsqrt(var + _EPS)
    inv = 1.0 / float(c_real)
    mean = jnp.sum(x, axis=-1, keepdims=True) * inv
    var = jnp.sum(x * x, axis=-1, keepdims=True) * inv - mean * mean
    r = jax.lax.rsqrt(var + _EPS)
    if masked_out:
        lane = jax.lax.broadcasted_iota(jnp.int32, x.shape, x.ndim - 1)
        mask = (lane < c_real).astype(jnp.float32)
        return (x - mean) * (r * mask)
    return (x - mean) * r


def _gelu2(u):
    return u + u * jax.lax.erf(u)


def _mlp_residual(t, w1, b1, w2h, b2, c):
    h = _ln(t, c, masked_out=False)
    h = jnp.dot(h, w1, preferred_element_type=jnp.float32) + b1
    h = _gelu2(h)
    h = jnp.dot(h, w2h, preferred_element_type=jnp.float32) + b2
    return t + h


def _merge(t, g, grid_hw, mw, mb, c_out):
    cp = t.shape[-1]
    half_rows = g * (grid_hw // 2) ** 2
    z = t.reshape(g, grid_hw // 2, 2, grid_hw, cp)
    e = z[:, :, 0].reshape(half_rows, 2 * cp)
    o = z[:, :, 1].reshape(half_rows, 2 * cp)
    y = (jnp.dot(e, mw[: 2 * cp], preferred_element_type=jnp.float32)
         + jnp.dot(o, mw[2 * cp:], preferred_element_type=jnp.float32) + mb)
    return _ln(y, c_out)


def _backbone_kernel(g, x_ref, ew_ref, eb_ref,
                     w10_ref, b10_ref, w20_ref, b20_ref, mw0_ref, mb0_ref,
                     w11_ref, b11_ref, w21_ref, b21_ref, mw1_ref, mb1_ref,
                     w12_ref, b12_ref, w22_ref, b22_ref, mw2_ref, mb2_ref,
                     w13_ref, b13_ref, w23_ref, b23_ref,
                     o0_ref, o1_ref, o2_ref, o3_ref):
    xp = x_ref[...].reshape(g * 3136, 48)
    y = jnp.dot(xp, ew_ref[...], preferred_element_type=jnp.float32) + eb_ref[...]
    t = _ln(y, 96)
    t = _mlp_residual(t, w10_ref[...], b10_ref[...], w20_ref[...], b20_ref[...], 96)

    t = _merge(t, g, 56, mw0_ref[...], mb0_ref[...], 192)
    t = _mlp_residual(t, w11_ref[...], b11_ref[...], w21_ref[...], b21_ref[...], 192)

    t = _merge(t, g, 28, mw1_ref[...], mb1_ref[...], 384)
    t = _mlp_residual(t, w12_ref[...], b12_ref[...], w22_ref[...], b22_ref[...], 384)

    n = _ln(t, 384)
    o0_ref[...] = n.reshape(g, 196, 384).astype(o0_ref.dtype)
    a = n.reshape(g, 7, 2, 14, 384)
    r = jnp.maximum(a[:, :, 0], a[:, :, 1])
    b4 = r.reshape(g, 7, 7, 2, 384)
    p = jnp.maximum(b4[:, :, :, 0], b4[:, :, :, 1])
    o1_ref[...] = p.reshape(g, 49, 384).astype(o1_ref.dtype)

    t = _merge(t, g, 14, mw2_ref[...], mb2_ref[...], 768)
    t = _mlp_residual(t, w13_ref[...], b13_ref[...], w23_ref[...], b23_ref[...], 768)

    n = _ln(t, 768).reshape(g, 49, 768)
    o2_ref[...] = n.astype(o2_ref.dtype)
    o3_ref[...] = jnp.max(n, axis=1, keepdims=True).astype(o3_ref.dtype)


def _const_spec(shape):
    nd = len(shape)
    return pl.BlockSpec(shape, lambda i: (0,) * nd)


def kernel(x, embed_w, embed_b,
           s0_fc1_w, s0_fc1_b, s0_fc2_w, s0_fc2_b, s0_merge_w, s0_merge_b,
           s1_fc1_w, s1_fc1_b, s1_fc2_w, s1_fc2_b, s1_merge_w, s1_merge_b,
           s2_fc1_w, s2_fc1_b, s2_fc2_w, s2_fc2_b, s2_merge_w, s2_merge_b,
           s3_fc1_w, s3_fc1_b, s3_fc2_w, s3_fc2_b):
    B = x.shape[0]
    f32 = jnp.float32
    g = 4

    xp = x.reshape(B, 3, 56, 4, 56, 4)
    xp = jnp.transpose(xp, (0, 2, 4, 3, 5, 1)).reshape(B, 3136, 48)

    wz = lambda w: w
    bz = lambda b: b.reshape(1, -1).astype(f32)

    def bzp(b, c):
        b = b.reshape(1, -1).astype(f32)
        lane = jax.lax.broadcasted_iota(jnp.int32, b.shape, 1)
        return jnp.where(lane < c, b, 0.0)

    rs2 = 0.7071067811865476
    wr = lambda w: (rs2 * w.astype(f32))
    br = lambda b: (rs2 * b.reshape(1, -1).astype(f32))
    half = lambda w: (rs2 * w.astype(f32))
    img = lambda n, c: pl.BlockSpec((g, n, c), lambda i: (i, 0, 0))

    weights = [
        (wz(embed_w), (48, 128)), (bzp(embed_b, 96), (1, 128)),
        (wr(s0_fc1_w), (128, 256)), (br(s0_fc1_b), (1, 256)),
        (half(s0_fc2_w), (256, 128)), (bzp(s0_fc2_b, 96), (1, 128)),
        (wz(s0_merge_w), (512, 256)), (bzp(s0_merge_b, 192), (1, 256)),
        (wr(s1_fc1_w), (256, 384)), (br(s1_fc1_b), (1, 384)),
        (half(s1_fc2_w), (384, 256)), (bzp(s1_fc2_b, 192), (1, 256)),
        (wz(s1_merge_w), (1024, 384)), (bz(s1_merge_b), (1, 384)),
        (wr(s2_fc1_w), (384, 768)), (br(s2_fc1_b), (1, 768)),
        (half(s2_fc2_w), (768, 384)), (bz(s2_fc2_b), (1, 384)),
        (wz(s2_merge_w), (1536, 768)), (bz(s2_merge_b), (1, 768)),
        (wr(s3_fc1_w), (768, 1536)), (br(s3_fc1_b), (1, 1536)),
        (half(s3_fc2_w), (1536, 768)), (bz(s3_fc2_b), (1, 768)),
    ]

    o0, o1, o2, o3 = pl.pallas_call(
        lambda *a: _backbone_kernel(g, *a),
        out_shape=(jax.ShapeDtypeStruct((B, 196, 384), f32),
                   jax.ShapeDtypeStruct((B, 49, 384), f32),
                   jax.ShapeDtypeStruct((B, 49, 768), f32),
                   jax.ShapeDtypeStruct((B, 1, 768), f32)),
        grid=(B // g,),
        in_specs=[img(3136, 48)] + [_const_spec(s) for _, s in weights],
        out_specs=(img(196, 384), img(49, 384), img(49, 768), img(1, 768)),
        compiler_params=pltpu.CompilerParams(
            dimension_semantics=("parallel",)),
    )(xp, *[w for w, _ in weights])

    return [[o0, o1], [o2, o3]]


# speedup vs baseline: 1.1049x; 1.0005x over previous
import jax
import jax.numpy as jnp
from jax.experimental import pallas as pl
from jax.experimental.pallas import tpu as pltpu

_EPS = 1e-5


def _ln(x, c_real, masked_out=True):
    cpad = x.shape[-1]
    if c_real == cpad:
        mean = jnp.mean(x, axis=-1, keepdims=True)
        xc = x - mean
        var = jnp.mean(xc * xc, axis=-1, keepdims=True)
        return xc * jax.lax.rsqrt(var + _EPS)
    inv = 1.0 / float(c_real)
    mean = jnp.sum(x, axis=-1, keepdims=True) * inv
    var = jnp.sum(x * x, axis=-1, keepdims=True) * inv - mean * mean
    r = jax.lax.rsqrt(var + _EPS)
    if masked_out:
        lane = jax.lax.broadcasted_iota(jnp.int32, x.shape, x.ndim - 1)
        mask = (lane < c_real).astype(jnp.float32)
        return (x - mean) * (r * mask)
    return (x - mean) * r


def _gelu2(u):
    return u + u * jax.lax.erf(u)


def _mlp_residual(t, w1, b1, w2h, b2, c):
    h = _ln(t, c, masked_out=False)
    h = jnp.dot(h, w1, preferred_element_type=jnp.float32) + b1
    h = _gelu2(h)
    h = jnp.dot(h, w2h, preferred_element_type=jnp.float32) + b2
    return t + h


def _merge(t, g, grid_hw, mw, mb, c_out):
    cp = t.shape[-1]
    half_rows = g * (grid_hw // 2) ** 2
    z = t.reshape(g, grid_hw // 2, 2, grid_hw, cp)
    e = z[:, :, 0].reshape(half_rows, 2 * cp)
    o = z[:, :, 1].reshape(half_rows, 2 * cp)
    y = (jnp.dot(e, mw[: 2 * cp], preferred_element_type=jnp.float32)
         + jnp.dot(o, mw[2 * cp:], preferred_element_type=jnp.float32) + mb)
    return _ln(y, c_out)


def _backbone_kernel(g, x_ref, ew_ref, eb_ref,
                     w10_ref, b10_ref, w20_ref, b20_ref, mw0_ref, mb0_ref,
                     w11_ref, b11_ref, w21_ref, b21_ref, mw1_ref, mb1_ref,
                     w12_ref, b12_ref, w22_ref, b22_ref, mw2_ref, mb2_ref,
                     w13_ref, b13_ref, w23_ref, b23_ref,
                     o0_ref, o1_ref, o2_ref, o3_ref):
    xp = x_ref[...].reshape(g * 3136, 48)
    y = jnp.dot(xp, ew_ref[...], preferred_element_type=jnp.float32) + eb_ref[...]
    t = _ln(y, 96)
    t = _mlp_residual(t, w10_ref[...], b10_ref[...], w20_ref[...], b20_ref[...], 96)

    t = _merge(t, g, 56, mw0_ref[...], mb0_ref[...], 192)
    t = _mlp_residual(t, w11_ref[...], b11_ref[...], w21_ref[...], b21_ref[...], 192)

    t = _merge(t, g, 28, mw1_ref[...], mb1_ref[...], 384)
    t = _mlp_residual(t, w12_ref[...], b12_ref[...], w22_ref[...], b22_ref[...], 384)

    n = _ln(t, 384)
    o0_ref[...] = n.reshape(g, 196, 384).astype(o0_ref.dtype)
    a = n.reshape(g, 7, 2, 14, 384)
    r = jnp.maximum(a[:, :, 0], a[:, :, 1])
    b4 = r.reshape(g, 7, 7, 2, 384)
    p = jnp.maximum(b4[:, :, :, 0], b4[:, :, :, 1])
    o1_ref[...] = p.reshape(g, 49, 384).astype(o1_ref.dtype)

    t = _merge(t, g, 14, mw2_ref[...], mb2_ref[...], 768)
    t = _mlp_residual(t, w13_ref[...], b13_ref[...], w23_ref[...], b23_ref[...], 768)

    n = _ln(t, 768).reshape(g, 49, 768)
    o2_ref[...] = n.astype(o2_ref.dtype)
    o3_ref[...] = jnp.max(n, axis=1, keepdims=True).astype(o3_ref.dtype)


def _const_spec(shape):
    nd = len(shape)
    return pl.BlockSpec(shape, lambda i: (0,) * nd)


def kernel(x, embed_w, embed_b,
           s0_fc1_w, s0_fc1_b, s0_fc2_w, s0_fc2_b, s0_merge_w, s0_merge_b,
           s1_fc1_w, s1_fc1_b, s1_fc2_w, s1_fc2_b, s1_merge_w, s1_merge_b,
           s2_fc1_w, s2_fc1_b, s2_fc2_w, s2_fc2_b, s2_merge_w, s2_merge_b,
           s3_fc1_w, s3_fc1_b, s3_fc2_w, s3_fc2_b):
    B = x.shape[0]
    f32 = jnp.float32
    g = 4

    xp = x.reshape(B, 3, 56, 4, 56, 4)
    xp = jnp.transpose(xp, (0, 2, 4, 3, 5, 1)).reshape(B, 3136, 48)

    wz = lambda w: w
    bz = lambda b: b.reshape(1, -1).astype(f32)

    def bzp(b, c):
        b = b.reshape(1, -1).astype(f32)
        lane = jax.lax.broadcasted_iota(jnp.int32, b.shape, 1)
        return jnp.where(lane < c, b, 0.0)

    rs2 = 0.7071067811865476
    wr = lambda w: (rs2 * w.astype(f32))
    br = lambda b: (rs2 * b.reshape(1, -1).astype(f32))
    half = lambda w: (rs2 * w.astype(f32))
    img = lambda n, c: pl.BlockSpec((g, n, c), lambda i: (i, 0, 0))

    weights = [
        (wz(embed_w), (48, 128)), (bzp(embed_b, 96), (1, 128)),
        (wr(s0_fc1_w), (128, 256)), (br(s0_fc1_b), (1, 256)),
        (half(s0_fc2_w), (256, 128)), (bzp(s0_fc2_b, 96), (1, 128)),
        (wz(s0_merge_w), (512, 256)), (bzp(s0_merge_b, 192), (1, 256)),
        (wr(s1_fc1_w), (256, 384)), (br(s1_fc1_b), (1, 384)),
        (half(s1_fc2_w), (384, 256)), (bzp(s1_fc2_b, 192), (1, 256)),
        (wz(s1_merge_w), (1024, 384)), (bz(s1_merge_b), (1, 384)),
        (wr(s2_fc1_w), (384, 768)), (br(s2_fc1_b), (1, 768)),
        (half(s2_fc2_w), (768, 384)), (bz(s2_fc2_b), (1, 384)),
        (wz(s2_merge_w), (1536, 768)), (bz(s2_merge_b), (1, 768)),
        (wr(s3_fc1_w), (768, 1536)), (br(s3_fc1_b), (1, 1536)),
        (half(s3_fc2_w), (1536, 768)), (bz(s3_fc2_b), (1, 768)),
    ]

    o0, o1, o2, o3 = pl.pallas_call(
        lambda *a: _backbone_kernel(g, *a),
        out_shape=(jax.ShapeDtypeStruct((B, 196, 384), f32),
                   jax.ShapeDtypeStruct((B, 49, 384), f32),
                   jax.ShapeDtypeStruct((B, 49, 768), f32),
                   jax.ShapeDtypeStruct((B, 1, 768), f32)),
        grid=(B // g,),
        in_specs=[img(3136, 48)] + [_const_spec(s) for _, s in weights],
        out_specs=(img(196, 384), img(49, 384), img(49, 768), img(1, 768)),
        compiler_params=pltpu.CompilerParams(
            dimension_semantics=("parallel",),
            allow_input_fusion=[True] + [False] * 24),
    )(xp, *[w for w, _ in weights])

    return [[o0, o1], [o2, o3]]
